# baseline (speedup 1.0000x reference)
import functools

import jax
import jax.numpy as jnp
from jax import lax
from jax.experimental import pallas as pl
from jax.experimental.pallas import tpu as pltpu
from jax.experimental.pallas import tpu_sc as plsc

_NBUF = 4
_NB = 4096
_NS = 200
_D = 32


def _build_gather():
    info = plsc.get_sparse_core_info()
    NC, NS_sub = info.num_cores, info.num_subcores
    NW = NC * NS_sub
    b_per_w = (_NB // NW) * _NS
    nbuf = _NBUF
    n_groups = _NS // nbuf
    mesh = plsc.VectorSubcoreMesh(core_axis_name="c", subcore_axis_name="s")

    @functools.partial(
        pl.kernel,
        mesh=mesh,
        out_type=jax.ShapeDtypeStruct((_NS, _D // 8, NW, 8, 128),
                                      jnp.float32),
        compiler_params=pltpu.CompilerParams(use_tc_tiling_on_sc=False,
                                             needs_layout_passes=False),
        scratch_types=(
            [pltpu.VMEM((b_per_w,), jnp.int32),
             pltpu.VMEM((b_per_w,), jnp.int32),
             pltpu.VMEM((nbuf, 128, _D), jnp.float32),
             pltpu.VMEM((nbuf, 1, _D // 8, 1, 8, 128), jnp.float32)]
            + [pltpu.SemaphoreType.DMA] * (2 * nbuf)
        ),
    )
    def gather_kernel(table_hbm, idx_hbm, out_hbm, idx_all, idx_t,
                      rows_v, tile_v, *sems):
        gsems, osems = sems[:nbuf], sems[nbuf:]
        wid = lax.axis_index("s") * NC + lax.axis_index("c")
        base = pl.multiple_of(wid * b_per_w, 8)
        iota = lax.broadcasted_iota(jnp.int32, (16,), 0)

        pltpu.sync_copy(idx_hbm.at[pl.ds(base, b_per_w)], idx_all)

        def idx_tr(s, carry):
            for g in range(8):
                v = plsc.load_gather(idx_all, [(iota + 16 * g) * _NS + s])
                idx_t[pl.ds(s * 128 + 16 * g, 16)] = v
            return carry
        lax.fori_loop(0, _NS, idx_tr, 0)

        def gather_start(s, b):
            soff = pl.multiple_of(s * 128, 8)
            pltpu.async_copy(
                table_hbm.at[idx_t.at[pl.ds(soff, 128)]],
                rows_v.at[b], gsems[b])

        def gather_wait(b):
            pltpu.make_async_copy(
                table_hbm.at[idx_t.at[pl.ds(0, 128)]],
                rows_v.at[b], gsems[b]).wait()

        def out_start(s, b):
            pltpu.async_copy(
                tile_v.at[b],
                out_hbm.at[pl.ds(s, 1), :, pl.ds(wid, 1)], osems[b])

        def out_wait(b):
            pltpu.make_async_copy(
                tile_v.at[b],
                out_hbm.at[pl.ds(0, 1), :, pl.ds(0, 1)], osems[b]).wait()

        def transpose(b):
            def dr_body(dr, carry):
                for r in range(8):
                    d = dr * 8 + r
                    for g in range(8):
                        v = plsc.load_gather(
                            rows_v.at[b], [iota + 16 * g, iota * 0 + d])
                        tile_v[b, 0, dr, 0, r, pl.ds(16 * g, 16)] = v
                return carry
            lax.fori_loop(0, _D // 8, dr_body, 0)

        for b in range(nbuf):
            gather_start(b, b)

        def group(gi, carry):
            for b in range(nbuf):
                s = gi * nbuf + b
                gather_wait(b)
                @pl.when(gi >= 1)
                def _(b=b):
                    out_wait(b)
                transpose(b)
                @pl.when(gi < n_groups - 1)
                def _(s=s, b=b):
                    gather_start(s + nbuf, b)
                out_start(s, b)
            return carry

        lax.fori_loop(0, n_groups, group, 0)

        for b in range(nbuf):
            out_wait(b)

    return gather_kernel


@jax.jit
def kernel(indices, table):
    idx_flat = indices.reshape(-1).astype(jnp.int32)
    out5 = _build_gather()(table, idx_flat)
    out = out5.transpose(2, 4, 0, 1, 3).reshape(_NB, _NS, _D)
    return out


# speedup vs baseline: 1.0081x; 1.0081x over previous
import functools

import jax
import jax.numpy as jnp
from jax import lax
from jax.experimental import pallas as pl
from jax.experimental.pallas import tpu as pltpu
from jax.experimental.pallas import tpu_sc as plsc

_NBG = 3
_NBT = 2
_NB = 4096
_NS = 200
_D = 32
_SG = 4
_CHUNK = _SG * 128
_NCHUNKS = _NS // _SG


def _build_gather():
    info = plsc.get_sparse_core_info()
    NC, NS_sub = info.num_cores, info.num_subcores
    NW = NC * NS_sub
    b_per_w = (_NB // NW) * _NS
    mesh = plsc.VectorSubcoreMesh(core_axis_name="c", subcore_axis_name="s")

    @functools.partial(
        pl.kernel,
        mesh=mesh,
        out_type=jax.ShapeDtypeStruct((_NS, _D // 8, NW, 1024), jnp.float32),
        compiler_params=pltpu.CompilerParams(use_tc_tiling_on_sc=False,
                                             needs_layout_passes=False),
        scratch_types=(
            [pltpu.VMEM((16 * _NS,), jnp.int32),
             pltpu.VMEM((b_per_w,), jnp.int32),
             pltpu.VMEM((_NBG, _CHUNK, _D), jnp.float32),
             pltpu.VMEM((_NBT, _SG, _D // 8, 1, 1024), jnp.float32),
             pltpu.SemaphoreType.DMA((_NBG,)),
             pltpu.SemaphoreType.DMA((_NBT,))]
        ),
    )
    def gather_kernel(table_hbm, idx_hbm, out_hbm, idx_piece, idx_t,
                      rows_v, tile_v, gsem, osem):
        wid = lax.axis_index("s") * NC + lax.axis_index("c")
        base = pl.multiple_of(wid * b_per_w, 8)
        iota = lax.broadcasted_iota(jnp.int32, (16,), 0)

        for p in range(8):
            pltpu.sync_copy(
                idx_hbm.at[pl.ds(base + p * 16 * _NS, 16 * _NS)], idx_piece)

            def idx_tr(s, carry, p=p):
                v = plsc.load_gather(idx_piece, [iota * _NS + s])
                idx_t[pl.ds(s * 128 + 16 * p, 16)] = v
                return carry
            lax.fori_loop(0, _NS, idx_tr, 0)

        def gather_start(c, b):
            coff = pl.multiple_of(c * _CHUNK, 8)
            pltpu.async_copy(
                table_hbm.at[idx_t.at[pl.ds(coff, _CHUNK)]],
                rows_v.at[b], gsem.at[b])

        def gather_wait(b):
            pltpu.make_async_copy(
                table_hbm.at[idx_t.at[pl.ds(0, _CHUNK)]],
                rows_v.at[b], gsem.at[b]).wait()

        def out_start(c, b):
            pltpu.async_copy(
                tile_v.at[b],
                out_hbm.at[pl.ds(c * _SG, _SG), :, pl.ds(wid, 1)],
                osem.at[b])

        def out_wait(b):
            pltpu.make_async_copy(
                tile_v.at[b],
                out_hbm.at[pl.ds(0, _SG), :, pl.ds(0, 1)],
                osem.at[b]).wait()

        def transpose(bg, bt):
            rows_b = rows_v.at[bg]

            def sl_body(sl, carry):
                row0 = sl * 128
                for dr in range(_D // 8):
                    for r in range(8):
                        d = dr * 8 + r
                        for g in range(8):
                            v = plsc.load_gather(
                                rows_b,
                                [row0 + 16 * g + iota, iota * 0 + d])
                            tile_v[bt, sl, dr, 0,
                                   pl.ds(r * 128 + 16 * g, 16)] = v
                return carry
            lax.fori_loop(0, _SG, sl_body, 0)

        for b in range(_NBG):
            gather_start(b, b)

        def chunk_body(c, carry):
            bg = lax.rem(c, _NBG)
            bt = lax.rem(c, _NBT)
            gather_wait(bg)
            @pl.when(c >= _NBT)
            def _():
                out_wait(bt)
            transpose(bg, bt)
            @pl.when(c + _NBG < _NCHUNKS)
            def _():
                gather_start(c + _NBG, bg)
            out_start(c, bt)
            return carry

        lax.fori_loop(0, _NCHUNKS, chunk_body, 0)

        for b in range(_NBT):
            out_wait(b)

    return gather_kernel


@jax.jit
def kernel(indices, table):
    idx_flat = indices.reshape(-1).astype(jnp.int32)
    out4 = _build_gather()(table, idx_flat)
    out = out4.reshape(_NS, _D // 8, 32, 8, 128)
    out = out.transpose(2, 4, 0, 1, 3).reshape(_NB, _NS, _D)
    return out


# speedup vs baseline: 1.0164x; 1.0083x over previous
import functools

import jax
import jax.numpy as jnp
from jax import lax
from jax.experimental import pallas as pl
from jax.experimental.pallas import tpu as pltpu
from jax.experimental.pallas import tpu_sc as plsc

_NBG = 3
_NBT = 2
_NB = 4096
_NS = 200
_D = 32
_SG = 4
_CHUNK = _SG * 128
_NCHUNKS = _NS // _SG


def _build_gather():
    info = plsc.get_sparse_core_info()
    NC, NS_sub = info.num_cores, info.num_subcores
    NW = NC * NS_sub
    b_per_w = (_NB // NW) * _NS
    mesh = plsc.VectorSubcoreMesh(core_axis_name="c", subcore_axis_name="s")

    @functools.partial(
        pl.kernel,
        mesh=mesh,
        out_type=jax.ShapeDtypeStruct((_NS, _D // 8, NW, 1024), jnp.float32),
        compiler_params=pltpu.CompilerParams(use_tc_tiling_on_sc=False,
                                             needs_layout_passes=False),
        scratch_types=(
            [pltpu.VMEM((b_per_w,), jnp.int32),
             pltpu.VMEM((_NBG, _CHUNK, _D), jnp.float32),
             pltpu.VMEM((_NBT, _SG, _D // 8, 1, 1024), jnp.float32),
             pltpu.SemaphoreType.DMA((_NBG,)),
             pltpu.SemaphoreType.DMA((_NBT,))]
        ),
    )
    def gather_kernel(table_hbm, idx_hbm, out_hbm, idx_t,
                      rows_v, tile_v, gsem, osem):
        wid = lax.axis_index("s") * NC + lax.axis_index("c")
        iota = lax.broadcasted_iota(jnp.int32, (16,), 0)

        pltpu.sync_copy(idx_hbm.at[wid], idx_t)

        def gather_start(c, b):
            coff = pl.multiple_of(c * _CHUNK, 8)
            pltpu.async_copy(
                table_hbm.at[idx_t.at[pl.ds(coff, _CHUNK)]],
                rows_v.at[b], gsem.at[b])

        def gather_wait(b):
            pltpu.make_async_copy(
                table_hbm.at[idx_t.at[pl.ds(0, _CHUNK)]],
                rows_v.at[b], gsem.at[b]).wait()

        def out_start(c, b):
            pltpu.async_copy(
                tile_v.at[b],
                out_hbm.at[pl.ds(c * _SG, _SG), :, pl.ds(wid, 1)],
                osem.at[b])

        def out_wait(b):
            pltpu.make_async_copy(
                tile_v.at[b],
                out_hbm.at[pl.ds(0, _SG), :, pl.ds(0, 1)],
                osem.at[b]).wait()

        rv = [[sl * 128 + g * 16 + iota for g in range(8)]
              for sl in range(_SG)]
        cv = [iota * 0 + d for d in range(_D)]

        def transpose(bg, bt):
            rows_b = rows_v.at[bg]
            for sl in range(_SG):
                for dr in range(_D // 8):
                    for r in range(8):
                        d = dr * 8 + r
                        for g in range(8):
                            v = plsc.load_gather(rows_b, [rv[sl][g], cv[d]])
                            tile_v[bt, sl, dr, 0,
                                   pl.ds(r * 128 + 16 * g, 16)] = v

        for b in range(_NBG):
            gather_start(b, b)

        def chunk_body(c, carry):
            bg = lax.rem(c, _NBG)
            bt = lax.rem(c, _NBT)
            gather_wait(bg)
            @pl.when(c >= _NBT)
            def _():
                out_wait(bt)
            transpose(bg, bt)
            @pl.when(c + _NBG < _NCHUNKS)
            def _():
                gather_start(c + _NBG, bg)
            out_start(c, bt)
            return carry

        lax.fori_loop(0, _NCHUNKS, chunk_body, 0)

        for b in range(_NBT):
            out_wait(b)

    return gather_kernel


@jax.jit
def kernel(indices, table):
    idx_w = (indices.astype(jnp.int32).T
             .reshape(_NS, _NB // 128, 128)
             .transpose(1, 0, 2)
             .reshape(_NB // 128, _NS * 128))
    out4 = _build_gather()(table, idx_w)
    out = out4.reshape(_NS, _D // 8, 32, 8, 128)
    out = out.transpose(2, 4, 0, 1, 3).reshape(_NB, _NS, _D)
    return out
